# baseline (device time: 177357 ns/iter reference)
import jax
import jax.numpy as jnp
from jax import lax
from jax.experimental import pallas as pl
from jax.experimental.pallas import tpu as pltpu

N_DEV = 4
M = 4096
N = 2048
CHUNK = M // N_DEV
HALF = CHUNK // 2
NSUB = 8
SUB = HALF // NSUB


def kernel(x, w_mat, scale_x, scale_w):
    def body(x_ref, w_ref, sx_ref, sw_ref, out_ref,
             w16_ref, comm_cw, comm_ccw, stage_ref,
             send_cw, recv_cw, send_ccw, recv_ccw, stage_sems):
        me = lax.axis_index("i")
        left = lax.rem(me + N_DEV - 1, N_DEV)
        right = lax.rem(me + 1, N_DEV)

        scale = sx_ref[0] * sw_ref[0]
        w16_ref[...] = w_ref[...].astype(jnp.bfloat16)

        def partial_sub(c, off, j):
            xh = x_ref[pl.ds(c * CHUNK + off + j * SUB, SUB), :].astype(
                jnp.bfloat16)
            return jnp.dot(
                xh, w16_ref[...], preferred_element_type=jnp.float32
            ) * scale

        def rdma(buf, sems_s, sems_r, s, r, j, dst):
            return pltpu.make_async_remote_copy(
                src_ref=buf.at[s, pl.ds(j * SUB, SUB), :],
                dst_ref=buf.at[r, pl.ds(j * SUB, SUB), :],
                send_sem=sems_s.at[s, j],
                recv_sem=sems_r.at[r, j],
                device_id=(dst,),
                device_id_type=pl.DeviceIdType.MESH,
            )

        def cw(s, r, j):
            return rdma(comm_cw, send_cw, recv_cw, s, r, j, right)

        def ccw(s, r, j):
            return rdma(comm_ccw, send_ccw, recv_ccw, s, r, j, left)

        stage_started = [False, False]

        def store_half(dir_slot, val_f32, c, off):
            if stage_started[dir_slot]:
                pltpu.make_async_copy(
                    stage_ref.at[dir_slot], stage_ref.at[dir_slot],
                    stage_sems.at[dir_slot],
                ).wait()
            stage_ref[dir_slot, :, :] = val_f32
            pltpu.make_async_copy(
                stage_ref.at[dir_slot],
                out_ref.at[pl.ds(c * CHUNK + off, HALF), :],
                stage_sems.at[dir_slot],
            ).start()
            stage_started[dir_slot] = True

        p_cw = [None] * NSUB
        p_ccw = [None] * NSUB
        p_cw[0] = partial_sub(me, 0, 0)
        p_ccw[0] = partial_sub(me, HALF, 0)

        barrier_sem = pltpu.get_barrier_semaphore()
        for nbr in [left, right]:
            pl.semaphore_signal(
                barrier_sem, inc=1,
                device_id=(nbr,), device_id_type=pl.DeviceIdType.MESH,
            )
        pl.semaphore_wait(barrier_sem, 2)

        for h in range(N_DEV - 1):
            s = h % 2
            r = (h + 1) % 2
            for j in range(NSUB):
                if h == 0 and j > 0:
                    p_cw[j] = partial_sub(me, 0, j)
                    p_ccw[j] = partial_sub(me, HALF, j)
                if h > 0:
                    cw(s, s, j).wait_recv()
                    ccw(s, s, j).wait_recv()
                    p_cw[j] = p_cw[j] + comm_cw[
                        s, pl.ds(j * SUB, SUB), :].astype(jnp.float32)
                    p_ccw[j] = p_ccw[j] + comm_ccw[
                        s, pl.ds(j * SUB, SUB), :].astype(jnp.float32)
                if h >= 2:
                    cw(s, s, j).wait_send()
                    ccw(s, s, j).wait_send()
                comm_cw[s, pl.ds(j * SUB, SUB), :] = p_cw[j].astype(jnp.bfloat16)
                comm_ccw[s, pl.ds(j * SUB, SUB), :] = p_ccw[j].astype(jnp.bfloat16)
                cw(s, r, j).start()
                ccw(s, r, j).start()
            c_cw = lax.rem(me + N_DEV - h - 1, N_DEV)
            c_ccw = lax.rem(me + h + 1, N_DEV)
            p_cw = [partial_sub(c_cw, 0, j) for j in range(NSUB)]
            p_ccw = [partial_sub(c_ccw, HALF, j) for j in range(NSUB)]

        v_cw, v_ccw = [None] * NSUB, [None] * NSUB
        for j in range(NSUB):
            cw(1, 1, j).wait_recv()
            ccw(1, 1, j).wait_recv()
            v_cw[j] = p_cw[j] + comm_cw[
                1, pl.ds(j * SUB, SUB), :].astype(jnp.float32)
            v_ccw[j] = p_ccw[j] + comm_ccw[
                1, pl.ds(j * SUB, SUB), :].astype(jnp.float32)
            cw(1, 1, j).wait_send()
            ccw(1, 1, j).wait_send()
            comm_cw[1, pl.ds(j * SUB, SUB), :] = v_cw[j].astype(jnp.bfloat16)
            comm_ccw[1, pl.ds(j * SUB, SUB), :] = v_ccw[j].astype(jnp.bfloat16)
            cw(1, 0, j).start()
            ccw(1, 0, j).start()

        c_cw_fin = lax.rem(me + 1, N_DEV)
        c_ccw_fin = lax.rem(me + 3, N_DEV)
        store_half(0, jnp.concatenate(v_cw, axis=0), c_cw_fin, 0)
        store_half(1, jnp.concatenate(v_ccw, axis=0), c_ccw_fin, HALF)

        for g in range(1, N_DEV - 1):
            h = (N_DEV - 1) + g
            s = h % 2
            r = (h + 1) % 2
            for j in range(NSUB):
                cw(s, s, j).wait_recv()
                ccw(s, s, j).wait_recv()
                cw(s, s, j).wait_send()
                ccw(s, s, j).wait_send()
                cw(s, r, j).start()
                ccw(s, r, j).start()
            c_cw = lax.rem(me + 1 + N_DEV - g, N_DEV)
            c_ccw = lax.rem(me + 3 + g, N_DEV)
            store_half(0, comm_cw[s, :, :].astype(jnp.float32), c_cw, 0)
            store_half(1, comm_ccw[s, :, :].astype(jnp.float32), c_ccw, HALF)

        for j in range(NSUB):
            cw(0, 0, j).wait_recv()
            ccw(0, 0, j).wait_recv()
        c_last = lax.rem(me + 2, N_DEV)
        store_half(0, comm_cw[0, :, :].astype(jnp.float32), c_last, 0)
        store_half(1, comm_ccw[0, :, :].astype(jnp.float32), c_last, HALF)

        for j in range(NSUB):
            cw(0, 0, j).wait_send()
            ccw(0, 0, j).wait_send()
            cw(1, 1, j).wait_send()
            ccw(1, 1, j).wait_send()
        for d in range(2):
            pltpu.make_async_copy(
                stage_ref.at[d], stage_ref.at[d], stage_sems.at[d]
            ).wait()

    return pl.pallas_call(
        body,
        out_shape=jax.ShapeDtypeStruct((M, N), jnp.float32),
        in_specs=[
            pl.BlockSpec(memory_space=pltpu.MemorySpace.VMEM),
            pl.BlockSpec(memory_space=pltpu.MemorySpace.VMEM),
            pl.BlockSpec(memory_space=pltpu.MemorySpace.SMEM),
            pl.BlockSpec(memory_space=pltpu.MemorySpace.SMEM),
        ],
        out_specs=pl.BlockSpec(memory_space=pltpu.MemorySpace.HBM),
        scratch_shapes=[
            pltpu.VMEM((CHUNK, N), jnp.bfloat16),
            pltpu.VMEM((2, HALF, N), jnp.bfloat16),
            pltpu.VMEM((2, HALF, N), jnp.bfloat16),
            pltpu.VMEM((2, HALF, N), jnp.float32),
            pltpu.SemaphoreType.DMA((2, NSUB)),
            pltpu.SemaphoreType.DMA((2, NSUB)),
            pltpu.SemaphoreType.DMA((2, NSUB)),
            pltpu.SemaphoreType.DMA((2, NSUB)),
            pltpu.SemaphoreType.DMA((2,)),
        ],
        compiler_params=pltpu.CompilerParams(
            collective_id=0,
            vmem_limit_bytes=100 * 1024 * 1024,
            skip_device_barrier=True,
        ),
    )(x, w_mat, scale_x, scale_w)


# device time: 173966 ns/iter; 1.0195x vs baseline; 1.0195x over previous
import jax
import jax.numpy as jnp
from jax import lax
from jax.experimental import pallas as pl
from jax.experimental.pallas import tpu as pltpu

N_DEV = 4
M = 4096
N = 2048
CHUNK = M // N_DEV
HALF = CHUNK // 2
NSUB = 4
SUB = HALF // NSUB


def kernel(x, w_mat, scale_x, scale_w):
    def body(x_ref, w_ref, sx_ref, sw_ref, out_ref,
             w16_ref, comm_cw, comm_ccw, stage_ref,
             send_cw, recv_cw, send_ccw, recv_ccw, stage_sems):
        me = lax.axis_index("i")
        left = lax.rem(me + N_DEV - 1, N_DEV)
        right = lax.rem(me + 1, N_DEV)

        scale = sx_ref[0] * sw_ref[0]
        w16_ref[...] = w_ref[...].astype(jnp.bfloat16)

        def partial_sub(c, off, j):
            xh = x_ref[pl.ds(c * CHUNK + off + j * SUB, SUB), :].astype(
                jnp.bfloat16)
            return jnp.dot(
                xh, w16_ref[...], preferred_element_type=jnp.float32
            ) * scale

        def rdma(buf, sems_s, sems_r, s, r, j, dst):
            return pltpu.make_async_remote_copy(
                src_ref=buf.at[s, pl.ds(j * SUB, SUB), :],
                dst_ref=buf.at[r, pl.ds(j * SUB, SUB), :],
                send_sem=sems_s.at[s, j],
                recv_sem=sems_r.at[r, j],
                device_id=(dst,),
                device_id_type=pl.DeviceIdType.MESH,
            )

        def cw(s, r, j):
            return rdma(comm_cw, send_cw, recv_cw, s, r, j, right)

        def ccw(s, r, j):
            return rdma(comm_ccw, send_ccw, recv_ccw, s, r, j, left)

        stage_started = [False, False]

        def store_half(dir_slot, val_f32, c, off):
            if stage_started[dir_slot]:
                pltpu.make_async_copy(
                    stage_ref.at[dir_slot], stage_ref.at[dir_slot],
                    stage_sems.at[dir_slot],
                ).wait()
            stage_ref[dir_slot, :, :] = val_f32
            pltpu.make_async_copy(
                stage_ref.at[dir_slot],
                out_ref.at[pl.ds(c * CHUNK + off, HALF), :],
                stage_sems.at[dir_slot],
            ).start()
            stage_started[dir_slot] = True

        p_cw = [None] * NSUB
        p_ccw = [None] * NSUB
        p_cw[0] = partial_sub(me, 0, 0)
        p_ccw[0] = partial_sub(me, HALF, 0)

        barrier_sem = pltpu.get_barrier_semaphore()
        for nbr in [left, right]:
            pl.semaphore_signal(
                barrier_sem, inc=1,
                device_id=(nbr,), device_id_type=pl.DeviceIdType.MESH,
            )
        pl.semaphore_wait(barrier_sem, 2)

        for h in range(N_DEV - 1):
            s = h % 2
            r = (h + 1) % 2
            for j in range(NSUB):
                if h == 0 and j > 0:
                    p_cw[j] = partial_sub(me, 0, j)
                    p_ccw[j] = partial_sub(me, HALF, j)
                if h > 0:
                    cw(s, s, j).wait_recv()
                    ccw(s, s, j).wait_recv()
                    p_cw[j] = p_cw[j] + comm_cw[
                        s, pl.ds(j * SUB, SUB), :].astype(jnp.float32)
                    p_ccw[j] = p_ccw[j] + comm_ccw[
                        s, pl.ds(j * SUB, SUB), :].astype(jnp.float32)
                if h >= 2:
                    cw(s, s, j).wait_send()
                    ccw(s, s, j).wait_send()
                comm_cw[s, pl.ds(j * SUB, SUB), :] = p_cw[j].astype(jnp.bfloat16)
                comm_ccw[s, pl.ds(j * SUB, SUB), :] = p_ccw[j].astype(jnp.bfloat16)
                cw(s, r, j).start()
                ccw(s, r, j).start()
            c_cw = lax.rem(me + N_DEV - h - 1, N_DEV)
            c_ccw = lax.rem(me + h + 1, N_DEV)
            p_cw = [partial_sub(c_cw, 0, j) for j in range(NSUB)]
            p_ccw = [partial_sub(c_ccw, HALF, j) for j in range(NSUB)]

        v_cw, v_ccw = [None] * NSUB, [None] * NSUB
        for j in range(NSUB):
            cw(1, 1, j).wait_recv()
            ccw(1, 1, j).wait_recv()
            v_cw[j] = p_cw[j] + comm_cw[
                1, pl.ds(j * SUB, SUB), :].astype(jnp.float32)
            v_ccw[j] = p_ccw[j] + comm_ccw[
                1, pl.ds(j * SUB, SUB), :].astype(jnp.float32)
            cw(1, 1, j).wait_send()
            ccw(1, 1, j).wait_send()
            comm_cw[1, pl.ds(j * SUB, SUB), :] = v_cw[j].astype(jnp.bfloat16)
            comm_ccw[1, pl.ds(j * SUB, SUB), :] = v_ccw[j].astype(jnp.bfloat16)
            cw(1, 0, j).start()
            ccw(1, 0, j).start()

        c_cw_fin = lax.rem(me + 1, N_DEV)
        c_ccw_fin = lax.rem(me + 3, N_DEV)
        store_half(0, jnp.concatenate(v_cw, axis=0), c_cw_fin, 0)
        store_half(1, jnp.concatenate(v_ccw, axis=0), c_ccw_fin, HALF)

        for g in range(1, N_DEV - 1):
            h = (N_DEV - 1) + g
            s = h % 2
            r = (h + 1) % 2
            for j in range(NSUB):
                cw(s, s, j).wait_recv()
                ccw(s, s, j).wait_recv()
                cw(s, s, j).wait_send()
                ccw(s, s, j).wait_send()
                cw(s, r, j).start()
                ccw(s, r, j).start()
            c_cw = lax.rem(me + 1 + N_DEV - g, N_DEV)
            c_ccw = lax.rem(me + 3 + g, N_DEV)
            store_half(0, comm_cw[s, :, :].astype(jnp.float32), c_cw, 0)
            store_half(1, comm_ccw[s, :, :].astype(jnp.float32), c_ccw, HALF)

        c_last = lax.rem(me + 2, N_DEV)
        pending = []
        for j in range(NSUB):
            cw(0, 0, j).wait_recv()
            ccw(0, 0, j).wait_recv()
            for d, buf in ((0, comm_cw), (1, comm_ccw)):
                if j == 0 and stage_started[d]:
                    pltpu.make_async_copy(
                        stage_ref.at[d], stage_ref.at[d], stage_sems.at[d]
                    ).wait()
                stage_ref[d, pl.ds(j * SUB, SUB), :] = buf[
                    0, pl.ds(j * SUB, SUB), :].astype(jnp.float32)
                cp = pltpu.make_async_copy(
                    stage_ref.at[d, pl.ds(j * SUB, SUB), :],
                    out_ref.at[
                        pl.ds(c_last * CHUNK + d * HALF + j * SUB, SUB), :],
                    stage_sems.at[d],
                )
                cp.start()
                pending.append(cp)

        for j in range(NSUB):
            cw(0, 0, j).wait_send()
            ccw(0, 0, j).wait_send()
            cw(1, 1, j).wait_send()
            ccw(1, 1, j).wait_send()
        for cp in pending:
            cp.wait()

    return pl.pallas_call(
        body,
        out_shape=jax.ShapeDtypeStruct((M, N), jnp.float32),
        in_specs=[
            pl.BlockSpec(memory_space=pltpu.MemorySpace.VMEM),
            pl.BlockSpec(memory_space=pltpu.MemorySpace.VMEM),
            pl.BlockSpec(memory_space=pltpu.MemorySpace.SMEM),
            pl.BlockSpec(memory_space=pltpu.MemorySpace.SMEM),
        ],
        out_specs=pl.BlockSpec(memory_space=pltpu.MemorySpace.HBM),
        scratch_shapes=[
            pltpu.VMEM((CHUNK, N), jnp.bfloat16),
            pltpu.VMEM((2, HALF, N), jnp.bfloat16),
            pltpu.VMEM((2, HALF, N), jnp.bfloat16),
            pltpu.VMEM((2, HALF, N), jnp.float32),
            pltpu.SemaphoreType.DMA((2, NSUB)),
            pltpu.SemaphoreType.DMA((2, NSUB)),
            pltpu.SemaphoreType.DMA((2, NSUB)),
            pltpu.SemaphoreType.DMA((2, NSUB)),
            pltpu.SemaphoreType.DMA((2,)),
        ],
        compiler_params=pltpu.CompilerParams(
            collective_id=0,
            vmem_limit_bytes=100 * 1024 * 1024,
            skip_device_barrier=True,
        ),
    )(x, w_mat, scale_x, scale_w)
